# Initial kernel scaffold; baseline (speedup 1.0000x reference)
import functools

import jax
import jax.numpy as jnp
from jax import lax
from jax.experimental import pallas as pl
from jax.experimental.pallas import tpu as pltpu
from jax.experimental.pallas import tpu_sc as plsc

_INTERP = False

_NB1 = 256
_NB = 512
_NBO = 256


def _three_nn_body(u_ref, kt_ref, gi_ref, w_ref, *, m):
    b = pl.program_id(0)
    u = u_ref[0]
    kt = kt_ref[0]
    dx = u[:, 0:1] - kt[0:1, :]
    dy = u[:, 1:2] - kt[1:2, :]
    dz = u[:, 2:3] - kt[2:3, :]
    d = dx * dx + dy * dy + dz * dz
    iota = lax.broadcasted_iota(jnp.int32, d.shape, 1)
    idxs, dists = [], []
    for _ in range(3):
        mj = jnp.min(d, axis=1, keepdims=True)
        ij = jnp.min(jnp.where(d == mj, iota, m), axis=1, keepdims=True)
        d = jnp.where(iota == ij, jnp.float32(3.4e38), d)
        idxs.append(ij)
        dists.append(mj)
    dr = [1.0 / (dj + 1e-8) for dj in dists]
    norm = dr[0] + dr[1] + dr[2]
    gi_ref[0] = jnp.concatenate(idxs, axis=1) + b * m
    w_ref[0] = jnp.concatenate([r / norm for r in dr], axis=1)


def _three_nn(unknown, known_t):
    b, n, _ = unknown.shape
    m = known_t.shape[2]
    return pl.pallas_call(
        functools.partial(_three_nn_body, m=m),
        grid=(b, n // _NB1),
        in_specs=[
            pl.BlockSpec((1, _NB1, 3), lambda bb, i: (bb, i, 0)),
            pl.BlockSpec((1, 3, m), lambda bb, i: (bb, 0, 0)),
        ],
        out_specs=[
            pl.BlockSpec((1, _NB1, 3), lambda bb, i: (bb, i, 0)),
            pl.BlockSpec((1, _NB1, 3), lambda bb, i: (bb, i, 0)),
        ],
        out_shape=[
            jax.ShapeDtypeStruct((b, n, 3), jnp.int32),
            jax.ShapeDtypeStruct((b, n, 3), jnp.float32),
        ],
        interpret=_INTERP,
    )(unknown, known_t)


def _sc_interpolate(kf_rows, idx_flat, w_flat, num_pts, c2):
    nw = 32
    ppw = num_pts // nw
    cp = 32
    nchunks = ppw // cp
    mesh = plsc.VectorSubcoreMesh(core_axis_name="c", subcore_axis_name="s")

    @functools.partial(
        pl.kernel,
        mesh=mesh,
        out_type=jax.ShapeDtypeStruct((num_pts, c2), jnp.float32),
        scratch_types=[
            pltpu.VMEM((3 * cp,), jnp.int32),
            pltpu.VMEM((3 * cp,), jnp.float32),
            pltpu.VMEM((3 * cp, c2), jnp.float32),
            pltpu.VMEM((cp, c2), jnp.float32),
            pltpu.SemaphoreType.DMA,
        ],
    )
    def k(kf_hbm, idx_hbm, w_hbm, out_hbm, idx_v, w_v, rows_v, out_v, sem):
        wid = lax.axis_index("s") * 2 + lax.axis_index("c")
        pt_base = wid * ppw

        def chunk(ci, carry):
            pt0 = pt_base + ci * cp
            pltpu.sync_copy(idx_hbm.at[pl.ds(pt0 * 3, 3 * cp)], idx_v)
            pltpu.sync_copy(w_hbm.at[pl.ds(pt0 * 3, 3 * cp)], w_v)
            pltpu.async_copy(kf_hbm.at[idx_v], rows_v, sem).wait()

            def pt(p, acc):
                r0 = 3 * p
                w0 = plsc.load_gather(w_v, [jnp.full((16,), r0, jnp.int32)])
                w1 = plsc.load_gather(w_v, [jnp.full((16,), r0 + 1, jnp.int32)])
                w2 = plsc.load_gather(w_v, [jnp.full((16,), r0 + 2, jnp.int32)])
                for c in range(c2 // 16):
                    sl = pl.ds(c * 16, 16)
                    out_v[p, sl] = (w0 * rows_v[r0, sl]
                                    + w1 * rows_v[r0 + 1, sl]
                                    + w2 * rows_v[r0 + 2, sl])
                return acc

            lax.fori_loop(0, cp, pt, 0)
            pltpu.sync_copy(out_v, out_hbm.at[pl.ds(pt0, cp)])
            return carry

        lax.fori_loop(0, nchunks, chunk, 0)

    return k(kf_rows, idx_flat, w_flat)


def _conv1_body(x1_ref, x2_ref, wa_ref, wb_ref, y_ref, s_ref, q_ref):
    i = pl.program_id(0)
    y = (jnp.dot(x1_ref[...], wa_ref[...], preferred_element_type=jnp.float32)
         + jnp.dot(x2_ref[...], wb_ref[...], preferred_element_type=jnp.float32))
    y_ref[...] = y

    @pl.when(i == 0)
    def _():
        s_ref[...] = jnp.zeros_like(s_ref)
        q_ref[...] = jnp.zeros_like(q_ref)

    s_ref[...] += jnp.sum(y, axis=0, keepdims=True)
    q_ref[...] += jnp.sum(y * y, axis=0, keepdims=True)


def _conv1(interp, uf_pm, wa_t, wb_t):
    bn_, c2 = interp.shape
    c1 = uf_pm.shape[1]
    cout = wa_t.shape[1]
    return pl.pallas_call(
        _conv1_body,
        grid=(bn_ // _NB,),
        in_specs=[
            pl.BlockSpec((_NB, c2), lambda i: (i, 0)),
            pl.BlockSpec((_NB, c1), lambda i: (i, 0)),
            pl.BlockSpec((c2, cout), lambda i: (0, 0)),
            pl.BlockSpec((c1, cout), lambda i: (0, 0)),
        ],
        out_specs=[
            pl.BlockSpec((_NB, cout), lambda i: (i, 0)),
            pl.BlockSpec((1, cout), lambda i: (0, 0)),
            pl.BlockSpec((1, cout), lambda i: (0, 0)),
        ],
        out_shape=[
            jax.ShapeDtypeStruct((bn_, cout), jnp.float32),
            jax.ShapeDtypeStruct((1, cout), jnp.float32),
            jax.ShapeDtypeStruct((1, cout), jnp.float32),
        ],
        interpret=_INTERP,
    )(interp, uf_pm, wa_t, wb_t)


def _conv2_body(y_ref, s_ref, q_ref, g_ref, b_ref, wc_ref, y2_ref, s2_ref,
                q2_ref, *, cnt):
    i = pl.program_id(0)
    mean = s_ref[...] / cnt
    var = q_ref[...] / cnt - mean * mean
    inv = lax.rsqrt(var + 1e-5)
    z = jnp.maximum((y_ref[...] - mean) * (inv * g_ref[...]) + b_ref[...], 0.0)
    y2 = jnp.dot(z, wc_ref[...], preferred_element_type=jnp.float32)
    y2_ref[...] = y2

    @pl.when(i == 0)
    def _():
        s2_ref[...] = jnp.zeros_like(s2_ref)
        q2_ref[...] = jnp.zeros_like(q2_ref)

    s2_ref[...] += jnp.sum(y2, axis=0, keepdims=True)
    q2_ref[...] += jnp.sum(y2 * y2, axis=0, keepdims=True)


def _conv2(y1, s1, q1, g1, b1, wc_t):
    bn_, c = y1.shape
    cout = wc_t.shape[1]
    return pl.pallas_call(
        functools.partial(_conv2_body, cnt=float(bn_)),
        grid=(bn_ // _NB,),
        in_specs=[
            pl.BlockSpec((_NB, c), lambda i: (i, 0)),
            pl.BlockSpec((1, c), lambda i: (0, 0)),
            pl.BlockSpec((1, c), lambda i: (0, 0)),
            pl.BlockSpec((1, c), lambda i: (0, 0)),
            pl.BlockSpec((1, c), lambda i: (0, 0)),
            pl.BlockSpec((c, cout), lambda i: (0, 0)),
        ],
        out_specs=[
            pl.BlockSpec((_NB, cout), lambda i: (i, 0)),
            pl.BlockSpec((1, cout), lambda i: (0, 0)),
            pl.BlockSpec((1, cout), lambda i: (0, 0)),
        ],
        out_shape=[
            jax.ShapeDtypeStruct((bn_, cout), jnp.float32),
            jax.ShapeDtypeStruct((1, cout), jnp.float32),
            jax.ShapeDtypeStruct((1, cout), jnp.float32),
        ],
        interpret=_INTERP,
    )(y1, s1, q1, g1, b1, wc_t)


def _enc_body(y_ref, s_ref, q_ref, g_ref, b_ref, cwt_ref, cc_ref, sc_ref,
              f_ref, e_ref, sa_ref, *, cnt):
    i = pl.program_id(1)
    mean = s_ref[...] / cnt
    var = q_ref[...] / cnt - mean * mean
    inv = lax.rsqrt(var + 1e-5)
    x = jnp.maximum((y_ref[...] - mean) * (inv * g_ref[...]) + b_ref[...], 0.0)
    f_ref[...] = x
    xx = jnp.sum(x * x, axis=1, keepdims=True)
    xc = jnp.dot(x, cwt_ref[...], preferred_element_type=jnp.float32)
    sl = sc_ref[...] * (xx + cc_ref[...] - 2.0 * xc)
    mx = jnp.max(sl, axis=1, keepdims=True)
    ea = jnp.exp(sl - mx)
    a = ea / jnp.sum(ea, axis=1, keepdims=True)
    ep = lax.dot_general(a, x, (((0,), (0,)), ((), ())),
                         preferred_element_type=jnp.float32)
    ones = jnp.ones((a.shape[0], 8), jnp.float32)
    sap = lax.dot_general(a, ones, (((0,), (0,)), ((), ())),
                          preferred_element_type=jnp.float32)

    @pl.when(i == 0)
    def _():
        e_ref[...] = jnp.zeros_like(e_ref)
        sa_ref[...] = jnp.zeros_like(sa_ref)

    e_ref[...] += ep[None]
    sa_ref[...] += sap[None]


def _enc(y2, s2, q2, g2, b2, cw_t, cc_row, sc_row, b, n):
    c = y2.shape[1]
    kk = cw_t.shape[1]
    return pl.pallas_call(
        functools.partial(_enc_body, cnt=float(y2.shape[0])),
        grid=(b, n // _NB),
        in_specs=[
            pl.BlockSpec((_NB, c), lambda bb, i: (bb * (n // _NB) + i, 0)),
            pl.BlockSpec((1, c), lambda bb, i: (0, 0)),
            pl.BlockSpec((1, c), lambda bb, i: (0, 0)),
            pl.BlockSpec((1, c), lambda bb, i: (0, 0)),
            pl.BlockSpec((1, c), lambda bb, i: (0, 0)),
            pl.BlockSpec((c, kk), lambda bb, i: (0, 0)),
            pl.BlockSpec((1, kk), lambda bb, i: (0, 0)),
            pl.BlockSpec((1, kk), lambda bb, i: (0, 0)),
        ],
        out_specs=[
            pl.BlockSpec((_NB, c), lambda bb, i: (bb * (n // _NB) + i, 0)),
            pl.BlockSpec((1, kk, c), lambda bb, i: (bb, 0, 0)),
            pl.BlockSpec((1, kk, 8), lambda bb, i: (bb, 0, 0)),
        ],
        out_shape=[
            jax.ShapeDtypeStruct((b * n, c), jnp.float32),
            jax.ShapeDtypeStruct((b, kk, c), jnp.float32),
            jax.ShapeDtypeStruct((b, kk, 8), jnp.float32),
        ],
        interpret=_INTERP,
    )(y2, s2, q2, g2, b2, cw_t, cc_row, sc_row)


def _ctx_body(e_ref, sa_ref, cw_ref, w3_ref, lb_ref, o_ref, es_ref, *, kk):
    e = jnp.maximum(e_ref[0] - sa_ref[0][:, 0:1] * cw_ref[...], 0.0)
    es_ref[...] = e
    nrm = jnp.sqrt(jnp.sum(e * e))
    sc = 1.0 / jnp.maximum(nrm, 1e-12)

    def kb(k, acc):
        ek = es_ref[pl.ds(k, 1), :]
        wk = w3_ref[k]
        return acc + jnp.dot(ek, wk, preferred_element_type=jnp.float32)

    acc = lax.fori_loop(0, kk, kb,
                        jnp.zeros((1, e.shape[1]), jnp.float32))
    z = acc * sc + lb_ref[...]
    o_ref[...] = 1.0 / (1.0 + jnp.exp(-z))


def _context(e_acc, sa_acc, cw, w3, lb_row):
    b, kk, c = e_acc.shape
    return pl.pallas_call(
        functools.partial(_ctx_body, kk=kk),
        grid=(b,),
        in_specs=[
            pl.BlockSpec((1, kk, c), lambda bb: (bb, 0, 0)),
            pl.BlockSpec((1, kk, 8), lambda bb: (bb, 0, 0)),
            pl.BlockSpec((kk, c), lambda bb: (0, 0)),
            pl.BlockSpec((kk, c, c), lambda bb: (0, 0, 0)),
            pl.BlockSpec((1, c), lambda bb: (0, 0)),
        ],
        out_specs=pl.BlockSpec((1, c), lambda bb: (bb, 0)),
        out_shape=jax.ShapeDtypeStruct((b, c), jnp.float32),
        scratch_shapes=[pltpu.VMEM((kk, c), jnp.float32)],
        interpret=_INTERP,
    )(e_acc, sa_acc, cw, w3, lb_row)


def _fin_body(f_ref, c_ref, o_ref):
    scaled = f_ref[...] * c_ref[...]
    o_ref[0] = scaled.T


def _finalize(feats, ctx, b, n):
    c = feats.shape[1]
    return pl.pallas_call(
        _fin_body,
        grid=(b, n // _NBO),
        in_specs=[
            pl.BlockSpec((_NBO, c), lambda bb, i: (bb * (n // _NBO) + i, 0)),
            pl.BlockSpec((1, c), lambda bb, i: (bb, 0)),
        ],
        out_specs=pl.BlockSpec((1, c, _NBO), lambda bb, i: (bb, 0, i)),
        out_shape=jax.ShapeDtypeStruct((b, c, n), jnp.float32),
        interpret=_INTERP,
    )(feats, ctx)


def kernel(unknown, known, unknow_feats, known_feats, conv1_w, bn1_g, bn1_b,
           conv2_w, bn2_g, bn2_b, codewords, scale, lin_w, lin_b):
    b, n, _ = unknown.shape
    m = known.shape[1]
    c1 = unknow_feats.shape[1]
    c2 = known_feats.shape[1]
    dd = conv2_w.shape[0]
    kk = codewords.shape[0]

    known_t = jnp.transpose(known, (0, 2, 1))
    kf_rows = jnp.transpose(known_feats, (0, 2, 1)).reshape(b * m, c2)
    uf_pm = jnp.transpose(unknow_feats, (0, 2, 1)).reshape(b * n, c1)

    gi, w3 = _three_nn(unknown, known_t)
    interp = _sc_interpolate(kf_rows, gi.reshape(b * n * 3),
                             w3.reshape(b * n * 3), b * n, c2)

    y1, s1, q1 = _conv1(interp, uf_pm, conv1_w[:, :c2].T, conv1_w[:, c2:].T)
    y2, s2, q2 = _conv2(y1, s1, q1, bn1_g.reshape(1, -1), bn1_b.reshape(1, -1),
                        conv2_w.T)
    cc_row = jnp.sum(codewords * codewords, axis=1).reshape(1, -1)
    feats, e_acc, sa_acc = _enc(y2, s2, q2, bn2_g.reshape(1, -1),
                                bn2_b.reshape(1, -1), codewords.T, cc_row,
                                scale.reshape(1, -1), b, n)
    w3lin = lin_w.reshape(dd, kk, dd).transpose(1, 2, 0)
    ctx = _context(e_acc, sa_acc, codewords, w3lin, lin_b.reshape(1, -1))
    return _finalize(feats, ctx, b, n)


# speedup vs baseline: 9.4765x; 9.4765x over previous
import functools

import jax
import jax.numpy as jnp
from jax import lax
from jax.experimental import pallas as pl
from jax.experimental.pallas import tpu as pltpu
from jax.experimental.pallas import tpu_sc as plsc

_INTERP = False

_NB1 = 256
_NB = 512
_NBO = 256


def _three_nn_body(u_ref, kt_ref, gi_ref, w_ref, *, m):
    b = pl.program_id(0)
    u = u_ref[0]
    kt = kt_ref[0]
    dx = u[:, 0:1] - kt[0:1, :]
    dy = u[:, 1:2] - kt[1:2, :]
    dz = u[:, 2:3] - kt[2:3, :]
    d = dx * dx + dy * dy + dz * dz
    iota = lax.broadcasted_iota(jnp.int32, d.shape, 1)
    idxs, dists = [], []
    for _ in range(3):
        mj = jnp.min(d, axis=1, keepdims=True)
        ij = jnp.min(jnp.where(d == mj, iota, m), axis=1, keepdims=True)
        d = jnp.where(iota == ij, jnp.float32(3.4e38), d)
        idxs.append(ij)
        dists.append(mj)
    dr = [1.0 / (dj + 1e-8) for dj in dists]
    norm = dr[0] + dr[1] + dr[2]
    gi_ref[0] = jnp.concatenate(idxs, axis=1) + b * m
    w_ref[0] = jnp.concatenate([r / norm for r in dr], axis=1)


def _three_nn(unknown, known_t):
    b, n, _ = unknown.shape
    m = known_t.shape[2]
    return pl.pallas_call(
        functools.partial(_three_nn_body, m=m),
        grid=(b, n // _NB1),
        in_specs=[
            pl.BlockSpec((1, _NB1, 3), lambda bb, i: (bb, i, 0)),
            pl.BlockSpec((1, 3, m), lambda bb, i: (bb, 0, 0)),
        ],
        out_specs=[
            pl.BlockSpec((1, _NB1, 3), lambda bb, i: (bb, i, 0)),
            pl.BlockSpec((1, _NB1, 3), lambda bb, i: (bb, i, 0)),
        ],
        out_shape=[
            jax.ShapeDtypeStruct((b, n, 3), jnp.int32),
            jax.ShapeDtypeStruct((b, n, 3), jnp.float32),
        ],
        interpret=_INTERP,
    )(unknown, known_t)


def _sc_interpolate(kf_rows, idx_flat, w_flat, num_pts, c2):
    nw = 32
    ppw = num_pts // nw
    cp = 32
    nchunks = ppw // cp
    mesh = plsc.VectorSubcoreMesh(core_axis_name="c", subcore_axis_name="s")

    @functools.partial(
        pl.kernel,
        mesh=mesh,
        out_type=jax.ShapeDtypeStruct((num_pts, c2), jnp.float32),
        scratch_types=[
            pltpu.VMEM((3 * cp,), jnp.int32),
            pltpu.VMEM((3 * cp + 16,), jnp.float32),
            pltpu.VMEM((3 * cp, c2), jnp.float32),
            pltpu.VMEM((cp, c2), jnp.float32),
            pltpu.SemaphoreType.DMA,
        ],
    )
    def k(kf_hbm, idx_hbm, w_hbm, out_hbm, idx_v, w_s, rows_v, out_v, sem):
        wid = lax.axis_index("s") * 2 + lax.axis_index("c")
        pt_base = wid * ppw

        def chunk(ci, carry):
            pt0 = pt_base + ci * cp
            pltpu.sync_copy(idx_hbm.at[pl.ds(pt0 * 3, 3 * cp)], idx_v)
            pltpu.sync_copy(w_hbm.at[pl.ds(pt0 * 3, 3 * cp)],
                            w_s.at[pl.ds(0, 3 * cp)])
            pltpu.async_copy(kf_hbm.at[idx_v], rows_v, sem).wait()

            def pt(p, acc):
                r0 = 3 * p
                wv = w_s[pl.ds(r0, 16)]
                w0 = wv[0]
                w1 = wv[1]
                w2 = wv[2]
                for c in range(c2 // 16):
                    sl = pl.ds(c * 16, 16)
                    out_v[p, sl] = (w0 * rows_v[r0, sl]
                                    + w1 * rows_v[r0 + 1, sl]
                                    + w2 * rows_v[r0 + 2, sl])
                return acc

            lax.fori_loop(0, cp, pt, 0)
            pltpu.sync_copy(out_v, out_hbm.at[pl.ds(pt0, cp)])
            return carry

        lax.fori_loop(0, nchunks, chunk, 0)

    return k(kf_rows, idx_flat, w_flat)


def _conv1_body(x1_ref, x2_ref, wa_ref, wb_ref, y_ref, s_ref, q_ref):
    i = pl.program_id(0)
    y = (jnp.dot(x1_ref[...], wa_ref[...], preferred_element_type=jnp.float32)
         + jnp.dot(x2_ref[...], wb_ref[...], preferred_element_type=jnp.float32))
    y_ref[...] = y

    @pl.when(i == 0)
    def _():
        s_ref[...] = jnp.zeros_like(s_ref)
        q_ref[...] = jnp.zeros_like(q_ref)

    s_ref[...] += jnp.sum(y, axis=0, keepdims=True)
    q_ref[...] += jnp.sum(y * y, axis=0, keepdims=True)


def _conv1(interp, uf_pm, wa_t, wb_t):
    bn_, c2 = interp.shape
    c1 = uf_pm.shape[1]
    cout = wa_t.shape[1]
    return pl.pallas_call(
        _conv1_body,
        grid=(bn_ // _NB,),
        in_specs=[
            pl.BlockSpec((_NB, c2), lambda i: (i, 0)),
            pl.BlockSpec((_NB, c1), lambda i: (i, 0)),
            pl.BlockSpec((c2, cout), lambda i: (0, 0)),
            pl.BlockSpec((c1, cout), lambda i: (0, 0)),
        ],
        out_specs=[
            pl.BlockSpec((_NB, cout), lambda i: (i, 0)),
            pl.BlockSpec((1, cout), lambda i: (0, 0)),
            pl.BlockSpec((1, cout), lambda i: (0, 0)),
        ],
        out_shape=[
            jax.ShapeDtypeStruct((bn_, cout), jnp.float32),
            jax.ShapeDtypeStruct((1, cout), jnp.float32),
            jax.ShapeDtypeStruct((1, cout), jnp.float32),
        ],
        interpret=_INTERP,
    )(interp, uf_pm, wa_t, wb_t)


def _conv2_body(y_ref, s_ref, q_ref, g_ref, b_ref, wc_ref, y2_ref, s2_ref,
                q2_ref, *, cnt):
    i = pl.program_id(0)
    mean = s_ref[...] / cnt
    var = q_ref[...] / cnt - mean * mean
    inv = lax.rsqrt(var + 1e-5)
    z = jnp.maximum((y_ref[...] - mean) * (inv * g_ref[...]) + b_ref[...], 0.0)
    y2 = jnp.dot(z, wc_ref[...], preferred_element_type=jnp.float32)
    y2_ref[...] = y2

    @pl.when(i == 0)
    def _():
        s2_ref[...] = jnp.zeros_like(s2_ref)
        q2_ref[...] = jnp.zeros_like(q2_ref)

    s2_ref[...] += jnp.sum(y2, axis=0, keepdims=True)
    q2_ref[...] += jnp.sum(y2 * y2, axis=0, keepdims=True)


def _conv2(y1, s1, q1, g1, b1, wc_t):
    bn_, c = y1.shape
    cout = wc_t.shape[1]
    return pl.pallas_call(
        functools.partial(_conv2_body, cnt=float(bn_)),
        grid=(bn_ // _NB,),
        in_specs=[
            pl.BlockSpec((_NB, c), lambda i: (i, 0)),
            pl.BlockSpec((1, c), lambda i: (0, 0)),
            pl.BlockSpec((1, c), lambda i: (0, 0)),
            pl.BlockSpec((1, c), lambda i: (0, 0)),
            pl.BlockSpec((1, c), lambda i: (0, 0)),
            pl.BlockSpec((c, cout), lambda i: (0, 0)),
        ],
        out_specs=[
            pl.BlockSpec((_NB, cout), lambda i: (i, 0)),
            pl.BlockSpec((1, cout), lambda i: (0, 0)),
            pl.BlockSpec((1, cout), lambda i: (0, 0)),
        ],
        out_shape=[
            jax.ShapeDtypeStruct((bn_, cout), jnp.float32),
            jax.ShapeDtypeStruct((1, cout), jnp.float32),
            jax.ShapeDtypeStruct((1, cout), jnp.float32),
        ],
        interpret=_INTERP,
    )(y1, s1, q1, g1, b1, wc_t)


def _enc_body(y_ref, s_ref, q_ref, g_ref, b_ref, cwt_ref, cc_ref, sc_ref,
              f_ref, e_ref, sa_ref, *, cnt):
    i = pl.program_id(1)
    mean = s_ref[...] / cnt
    var = q_ref[...] / cnt - mean * mean
    inv = lax.rsqrt(var + 1e-5)
    x = jnp.maximum((y_ref[...] - mean) * (inv * g_ref[...]) + b_ref[...], 0.0)
    f_ref[...] = x
    xx = jnp.sum(x * x, axis=1, keepdims=True)
    xc = jnp.dot(x, cwt_ref[...], preferred_element_type=jnp.float32)
    sl = sc_ref[...] * (xx + cc_ref[...] - 2.0 * xc)
    mx = jnp.max(sl, axis=1, keepdims=True)
    ea = jnp.exp(sl - mx)
    a = ea / jnp.sum(ea, axis=1, keepdims=True)
    ep = lax.dot_general(a, x, (((0,), (0,)), ((), ())),
                         preferred_element_type=jnp.float32)
    ones = jnp.ones((a.shape[0], 8), jnp.float32)
    sap = lax.dot_general(a, ones, (((0,), (0,)), ((), ())),
                          preferred_element_type=jnp.float32)

    @pl.when(i == 0)
    def _():
        e_ref[...] = jnp.zeros_like(e_ref)
        sa_ref[...] = jnp.zeros_like(sa_ref)

    e_ref[...] += ep[None]
    sa_ref[...] += sap[None]


def _enc(y2, s2, q2, g2, b2, cw_t, cc_row, sc_row, b, n):
    c = y2.shape[1]
    kk = cw_t.shape[1]
    return pl.pallas_call(
        functools.partial(_enc_body, cnt=float(y2.shape[0])),
        grid=(b, n // _NB),
        in_specs=[
            pl.BlockSpec((_NB, c), lambda bb, i: (bb * (n // _NB) + i, 0)),
            pl.BlockSpec((1, c), lambda bb, i: (0, 0)),
            pl.BlockSpec((1, c), lambda bb, i: (0, 0)),
            pl.BlockSpec((1, c), lambda bb, i: (0, 0)),
            pl.BlockSpec((1, c), lambda bb, i: (0, 0)),
            pl.BlockSpec((c, kk), lambda bb, i: (0, 0)),
            pl.BlockSpec((1, kk), lambda bb, i: (0, 0)),
            pl.BlockSpec((1, kk), lambda bb, i: (0, 0)),
        ],
        out_specs=[
            pl.BlockSpec((_NB, c), lambda bb, i: (bb * (n // _NB) + i, 0)),
            pl.BlockSpec((1, kk, c), lambda bb, i: (bb, 0, 0)),
            pl.BlockSpec((1, kk, 8), lambda bb, i: (bb, 0, 0)),
        ],
        out_shape=[
            jax.ShapeDtypeStruct((b * n, c), jnp.float32),
            jax.ShapeDtypeStruct((b, kk, c), jnp.float32),
            jax.ShapeDtypeStruct((b, kk, 8), jnp.float32),
        ],
        interpret=_INTERP,
    )(y2, s2, q2, g2, b2, cw_t, cc_row, sc_row)


def _ctx_body(e_ref, sa_ref, cw_ref, w3_ref, lb_ref, o_ref, es_ref, *, kk):
    e = jnp.maximum(e_ref[0] - sa_ref[0][:, 0:1] * cw_ref[...], 0.0)
    es_ref[...] = e
    nrm = jnp.sqrt(jnp.sum(e * e))
    sc = 1.0 / jnp.maximum(nrm, 1e-12)

    def kb(k, acc):
        ek = es_ref[pl.ds(k, 1), :]
        wk = w3_ref[k]
        return acc + jnp.dot(ek, wk, preferred_element_type=jnp.float32)

    acc = lax.fori_loop(0, kk, kb,
                        jnp.zeros((1, e.shape[1]), jnp.float32))
    z = acc * sc + lb_ref[...]
    o_ref[0] = 1.0 / (1.0 + jnp.exp(-z))


def _context(e_acc, sa_acc, cw, w3, lb_row):
    b, kk, c = e_acc.shape
    return pl.pallas_call(
        functools.partial(_ctx_body, kk=kk),
        grid=(b,),
        in_specs=[
            pl.BlockSpec((1, kk, c), lambda bb: (bb, 0, 0)),
            pl.BlockSpec((1, kk, 8), lambda bb: (bb, 0, 0)),
            pl.BlockSpec((kk, c), lambda bb: (0, 0)),
            pl.BlockSpec((kk, c, c), lambda bb: (0, 0, 0)),
            pl.BlockSpec((1, c), lambda bb: (0, 0)),
        ],
        out_specs=pl.BlockSpec((1, 1, c), lambda bb: (bb, 0, 0)),
        out_shape=jax.ShapeDtypeStruct((b, 1, c), jnp.float32),
        scratch_shapes=[pltpu.VMEM((kk, c), jnp.float32)],
        interpret=_INTERP,
    )(e_acc, sa_acc, cw, w3, lb_row)


def _fin_body(f_ref, c_ref, o_ref):
    scaled = f_ref[...] * c_ref[0]
    o_ref[0] = scaled.T


def _finalize(feats, ctx, b, n):
    c = feats.shape[1]
    return pl.pallas_call(
        _fin_body,
        grid=(b, n // _NBO),
        in_specs=[
            pl.BlockSpec((_NBO, c), lambda bb, i: (bb * (n // _NBO) + i, 0)),
            pl.BlockSpec((1, 1, c), lambda bb, i: (bb, 0, 0)),
        ],
        out_specs=pl.BlockSpec((1, c, _NBO), lambda bb, i: (bb, 0, i)),
        out_shape=jax.ShapeDtypeStruct((b, c, n), jnp.float32),
        interpret=_INTERP,
    )(feats, ctx)


def kernel(unknown, known, unknow_feats, known_feats, conv1_w, bn1_g, bn1_b,
           conv2_w, bn2_g, bn2_b, codewords, scale, lin_w, lin_b):
    b, n, _ = unknown.shape
    m = known.shape[1]
    c1 = unknow_feats.shape[1]
    c2 = known_feats.shape[1]
    dd = conv2_w.shape[0]
    kk = codewords.shape[0]

    known_t = jnp.transpose(known, (0, 2, 1))
    kf_rows = jnp.transpose(known_feats, (0, 2, 1)).reshape(b * m, c2)
    uf_pm = jnp.transpose(unknow_feats, (0, 2, 1)).reshape(b * n, c1)

    gi, w3 = _three_nn(unknown, known_t)
    interp = _sc_interpolate(kf_rows, gi.reshape(b * n * 3),
                             w3.reshape(b * n * 3), b * n, c2)

    y1, s1, q1 = _conv1(interp, uf_pm, conv1_w[:, :c2].T, conv1_w[:, c2:].T)
    y2, s2, q2 = _conv2(y1, s1, q1, bn1_g.reshape(1, -1), bn1_b.reshape(1, -1),
                        conv2_w.T)
    cc_row = jnp.sum(codewords * codewords, axis=1).reshape(1, -1)
    feats, e_acc, sa_acc = _enc(y2, s2, q2, bn2_g.reshape(1, -1),
                                bn2_b.reshape(1, -1), codewords.T, cc_row,
                                scale.reshape(1, -1), b, n)
    w3lin = lin_w.reshape(dd, kk, dd).transpose(1, 2, 0)
    ctx = _context(e_acc, sa_acc, codewords, w3lin, lin_b.reshape(1, -1))
    return _finalize(feats, ctx, b, n)


# speedup vs baseline: 11.6913x; 1.2337x over previous
import functools

import jax
import jax.numpy as jnp
from jax import lax
from jax.experimental import pallas as pl
from jax.experimental.pallas import tpu as pltpu
from jax.experimental.pallas import tpu_sc as plsc

_INTERP = False

_NB1 = 256
_NB = 512
_NBO = 256


def _three_nn_body(u_ref, kt_ref, gi_ref, w_ref, *, m):
    b = pl.program_id(0)
    u = u_ref[0]
    kt = kt_ref[0]
    dx = u[:, 0:1] - kt[0:1, :]
    dy = u[:, 1:2] - kt[1:2, :]
    dz = u[:, 2:3] - kt[2:3, :]
    d = dx * dx + dy * dy + dz * dz
    iota_f = lax.broadcasted_iota(jnp.int32, d.shape, 1).astype(jnp.float32)
    idxs, dists = [], []
    for _ in range(3):
        mj = jnp.min(d, axis=1, keepdims=True)
        eq = d == mj
        ij = jnp.min(jnp.where(eq, iota_f, jnp.float32(m)),
                     axis=1, keepdims=True)
        d = jnp.where(eq, jnp.float32(3.4e38), d)
        idxs.append(ij)
        dists.append(mj)
    dr = [1.0 / (dj + 1e-8) for dj in dists]
    norm = dr[0] + dr[1] + dr[2]
    idx_t = jnp.transpose(jnp.concatenate(idxs, axis=1))
    w_t = jnp.transpose(jnp.concatenate([r / norm for r in dr], axis=1))
    gi_ref[...] = idx_t.astype(jnp.int32) + b * m
    w_ref[...] = w_t


def _three_nn(unknown, known_t):
    b, n, _ = unknown.shape
    m = known_t.shape[2]
    nblk = n // _NB1
    return pl.pallas_call(
        functools.partial(_three_nn_body, m=m),
        grid=(b, nblk),
        in_specs=[
            pl.BlockSpec((1, _NB1, 3), lambda bb, i: (bb, i, 0)),
            pl.BlockSpec((1, 3, m), lambda bb, i: (bb, 0, 0)),
        ],
        out_specs=[
            pl.BlockSpec((3, _NB1), lambda bb, i: (0, bb * nblk + i)),
            pl.BlockSpec((3, _NB1), lambda bb, i: (0, bb * nblk + i)),
        ],
        out_shape=[
            jax.ShapeDtypeStruct((3, b * n), jnp.int32),
            jax.ShapeDtypeStruct((3, b * n), jnp.float32),
        ],
        interpret=_INTERP,
    )(unknown, known_t)


def _sc_interpolate(kf_rows, idx_flat, w_flat, num_pts, c2):
    nw = 32
    ppw = num_pts // nw
    cp = 32
    nchunks = ppw // cp
    mesh = plsc.VectorSubcoreMesh(core_axis_name="c", subcore_axis_name="s")

    @functools.partial(
        pl.kernel,
        mesh=mesh,
        out_type=jax.ShapeDtypeStruct((num_pts, c2), jnp.float32),
        scratch_types=[
            pltpu.VMEM((3, ppw), jnp.int32),
            pltpu.VMEM((3, ppw + 16), jnp.float32),
            pltpu.VMEM((2, 3, cp, c2), jnp.float32),
            pltpu.VMEM((2, cp, c2), jnp.float32),
            pltpu.SemaphoreType.DMA,
            pltpu.SemaphoreType.DMA,
        ],
    )
    def k(kf_hbm, idx_hbm, w_hbm, out_hbm, idx_v, w_v, rows_v, out_v,
          sem_g, sem_o):
        wid = lax.axis_index("s") * 2 + lax.axis_index("c")
        pt_base = wid * ppw
        pltpu.sync_copy(idx_hbm.at[:, pl.ds(pt_base, ppw)], idx_v)
        pltpu.sync_copy(w_hbm.at[:, pl.ds(pt_base, ppw)],
                        w_v.at[:, pl.ds(0, ppw)])

        def start_gather(ci, slot):
            for j in range(3):
                pltpu.async_copy(
                    kf_hbm.at[idx_v.at[j, pl.ds(ci * cp, cp)]],
                    rows_v.at[slot, j], sem_g)

        def wait_gather(ci, slot):
            for j in range(3):
                pltpu.make_async_copy(
                    kf_hbm.at[idx_v.at[j, pl.ds(ci * cp, cp)]],
                    rows_v.at[slot, j], sem_g).wait()

        start_gather(0, 0)

        def pair(cb, carry):
            for b in range(2):
                ci = 2 * cb + b
                wait_gather(ci, b)

                @pl.when(ci + 1 < nchunks)
                def _():
                    start_gather(ci + 1, 1 - b)

                @pl.when(ci >= 2)
                def _():
                    pltpu.make_async_copy(
                        out_v.at[b], out_hbm.at[pl.ds(0, cp)], sem_o).wait()

                for g in range(0, cp, 16):
                    wv0 = w_v[0, pl.ds(ci * cp + g, 16)]
                    wv1 = w_v[1, pl.ds(ci * cp + g, 16)]
                    wv2 = w_v[2, pl.ds(ci * cp + g, 16)]
                    for e in range(16):
                        p = g + e
                        w0 = wv0[e]
                        w1 = wv1[e]
                        w2 = wv2[e]
                        for c in range(c2 // 16):
                            sl = pl.ds(c * 16, 16)
                            out_v[b, p, sl] = (w0 * rows_v[b, 0, p, sl]
                                               + w1 * rows_v[b, 1, p, sl]
                                               + w2 * rows_v[b, 2, p, sl])
                pltpu.async_copy(out_v.at[b],
                                 out_hbm.at[pl.ds(pt_base + ci * cp, cp)],
                                 sem_o)
            return carry

        lax.fori_loop(0, nchunks // 2, pair, 0)
        for b in range(2):
            pltpu.make_async_copy(out_v.at[b], out_hbm.at[pl.ds(0, cp)],
                                  sem_o).wait()

    return k(kf_rows, idx_flat, w_flat)


def _pre_body(uf_ref, wb_ref, p_ref):
    p_ref[...] = lax.dot_general(uf_ref[0], wb_ref[...],
                                 (((0,), (0,)), ((), ())),
                                 preferred_element_type=jnp.float32)


def _pre_conv1(uf, wb_t, b, n):
    c1 = uf.shape[1]
    cout = wb_t.shape[1]
    return pl.pallas_call(
        _pre_body,
        grid=(b, n // _NB),
        in_specs=[
            pl.BlockSpec((1, c1, _NB), lambda bb, i: (bb, 0, i)),
            pl.BlockSpec((c1, cout), lambda bb, i: (0, 0)),
        ],
        out_specs=pl.BlockSpec((_NB, cout),
                               lambda bb, i: (bb * (n // _NB) + i, 0)),
        out_shape=jax.ShapeDtypeStruct((b * n, cout), jnp.float32),
        interpret=_INTERP,
    )(uf, wb_t)


def _conv1_body(x1_ref, p_ref, wa_ref, y_ref, s_ref, q_ref):
    first = jnp.logical_and(pl.program_id(0) == 0, pl.program_id(1) == 0)
    y = (jnp.dot(x1_ref[...], wa_ref[...], preferred_element_type=jnp.float32)
         + p_ref[...])
    y_ref[...] = y

    @pl.when(first)
    def _():
        s_ref[...] = jnp.zeros_like(s_ref)
        q_ref[...] = jnp.zeros_like(q_ref)

    s_ref[...] += jnp.sum(y, axis=0, keepdims=True)
    q_ref[...] += jnp.sum(y * y, axis=0, keepdims=True)


def _conv1(interp, p1, wa_t, b, n):
    c2 = interp.shape[1]
    cout = wa_t.shape[1]
    return pl.pallas_call(
        _conv1_body,
        grid=(b, n // _NB),
        in_specs=[
            pl.BlockSpec((_NB, c2), lambda bb, i: (bb * (n // _NB) + i, 0)),
            pl.BlockSpec((_NB, cout), lambda bb, i: (bb * (n // _NB) + i, 0)),
            pl.BlockSpec((c2, cout), lambda bb, i: (0, 0)),
        ],
        out_specs=[
            pl.BlockSpec((_NB, cout), lambda bb, i: (bb * (n // _NB) + i, 0)),
            pl.BlockSpec((1, cout), lambda bb, i: (0, 0)),
            pl.BlockSpec((1, cout), lambda bb, i: (0, 0)),
        ],
        out_shape=[
            jax.ShapeDtypeStruct((b * n, cout), jnp.float32),
            jax.ShapeDtypeStruct((1, cout), jnp.float32),
            jax.ShapeDtypeStruct((1, cout), jnp.float32),
        ],
        interpret=_INTERP,
    )(interp, p1, wa_t)


def _conv2_body(y_ref, s_ref, q_ref, g_ref, b_ref, wc_ref, y2_ref, s2_ref,
                q2_ref, *, cnt):
    i = pl.program_id(0)
    mean = s_ref[...] / cnt
    var = q_ref[...] / cnt - mean * mean
    inv = lax.rsqrt(var + 1e-5)
    z = jnp.maximum((y_ref[...] - mean) * (inv * g_ref[...]) + b_ref[...], 0.0)
    y2 = jnp.dot(z, wc_ref[...], preferred_element_type=jnp.float32)
    y2_ref[...] = y2

    @pl.when(i == 0)
    def _():
        s2_ref[...] = jnp.zeros_like(s2_ref)
        q2_ref[...] = jnp.zeros_like(q2_ref)

    s2_ref[...] += jnp.sum(y2, axis=0, keepdims=True)
    q2_ref[...] += jnp.sum(y2 * y2, axis=0, keepdims=True)


def _conv2(y1, s1, q1, g1, b1, wc_t):
    bn_, c = y1.shape
    cout = wc_t.shape[1]
    return pl.pallas_call(
        functools.partial(_conv2_body, cnt=float(bn_)),
        grid=(bn_ // _NB,),
        in_specs=[
            pl.BlockSpec((_NB, c), lambda i: (i, 0)),
            pl.BlockSpec((1, c), lambda i: (0, 0)),
            pl.BlockSpec((1, c), lambda i: (0, 0)),
            pl.BlockSpec((1, c), lambda i: (0, 0)),
            pl.BlockSpec((1, c), lambda i: (0, 0)),
            pl.BlockSpec((c, cout), lambda i: (0, 0)),
        ],
        out_specs=[
            pl.BlockSpec((_NB, cout), lambda i: (i, 0)),
            pl.BlockSpec((1, cout), lambda i: (0, 0)),
            pl.BlockSpec((1, cout), lambda i: (0, 0)),
        ],
        out_shape=[
            jax.ShapeDtypeStruct((bn_, cout), jnp.float32),
            jax.ShapeDtypeStruct((1, cout), jnp.float32),
            jax.ShapeDtypeStruct((1, cout), jnp.float32),
        ],
        interpret=_INTERP,
    )(y1, s1, q1, g1, b1, wc_t)


def _enc_body(y_ref, s_ref, q_ref, g_ref, b_ref, cwt_ref, cc_ref, sc_ref,
              e_ref, sa_ref, *, cnt):
    i = pl.program_id(1)
    mean = s_ref[...] / cnt
    var = q_ref[...] / cnt - mean * mean
    inv = lax.rsqrt(var + 1e-5)
    x = jnp.maximum((y_ref[...] - mean) * (inv * g_ref[...]) + b_ref[...], 0.0)
    xx = jnp.sum(x * x, axis=1, keepdims=True)
    xc = jnp.dot(x, cwt_ref[...], preferred_element_type=jnp.float32)
    sl = sc_ref[...] * (xx + cc_ref[...] - 2.0 * xc)
    mx = jnp.max(sl, axis=1, keepdims=True)
    ea = jnp.exp(sl - mx)
    a = ea / jnp.sum(ea, axis=1, keepdims=True)
    ep = lax.dot_general(a, x, (((0,), (0,)), ((), ())),
                         preferred_element_type=jnp.float32)
    ones = jnp.ones((a.shape[0], 8), jnp.float32)
    sap = lax.dot_general(a, ones, (((0,), (0,)), ((), ())),
                          preferred_element_type=jnp.float32)

    @pl.when(i == 0)
    def _():
        e_ref[...] = jnp.zeros_like(e_ref)
        sa_ref[...] = jnp.zeros_like(sa_ref)

    e_ref[...] += ep[None]
    sa_ref[...] += sap[None]


def _enc(y2, s2, q2, g2, b2, cw_t, cc_row, sc_row, b, n):
    c = y2.shape[1]
    kk = cw_t.shape[1]
    return pl.pallas_call(
        functools.partial(_enc_body, cnt=float(y2.shape[0])),
        grid=(b, n // _NB),
        in_specs=[
            pl.BlockSpec((_NB, c), lambda bb, i: (bb * (n // _NB) + i, 0)),
            pl.BlockSpec((1, c), lambda bb, i: (0, 0)),
            pl.BlockSpec((1, c), lambda bb, i: (0, 0)),
            pl.BlockSpec((1, c), lambda bb, i: (0, 0)),
            pl.BlockSpec((1, c), lambda bb, i: (0, 0)),
            pl.BlockSpec((c, kk), lambda bb, i: (0, 0)),
            pl.BlockSpec((1, kk), lambda bb, i: (0, 0)),
            pl.BlockSpec((1, kk), lambda bb, i: (0, 0)),
        ],
        out_specs=[
            pl.BlockSpec((1, kk, c), lambda bb, i: (bb, 0, 0)),
            pl.BlockSpec((1, kk, 8), lambda bb, i: (bb, 0, 0)),
        ],
        out_shape=[
            jax.ShapeDtypeStruct((b, kk, c), jnp.float32),
            jax.ShapeDtypeStruct((b, kk, 8), jnp.float32),
        ],
        interpret=_INTERP,
    )(y2, s2, q2, g2, b2, cw_t, cc_row, sc_row)


def _ctx_body(e_ref, sa_ref, cw_ref, w3_ref, lb_ref, o_ref, es_ref, *, kk):
    e = jnp.maximum(e_ref[0] - sa_ref[0][:, 0:1] * cw_ref[...], 0.0)
    es_ref[...] = e
    nrm = jnp.sqrt(jnp.sum(e * e))
    sc = 1.0 / jnp.maximum(nrm, 1e-12)

    def kb(k, acc):
        ek = es_ref[pl.ds(k, 1), :]
        wk = w3_ref[k]
        return acc + jnp.dot(ek, wk, preferred_element_type=jnp.float32)

    acc = lax.fori_loop(0, kk, kb,
                        jnp.zeros((1, e.shape[1]), jnp.float32))
    z = acc * sc + lb_ref[...]
    o_ref[0] = 1.0 / (1.0 + jnp.exp(-z))


def _context(e_acc, sa_acc, cw, w3, lb_row):
    b, kk, c = e_acc.shape
    return pl.pallas_call(
        functools.partial(_ctx_body, kk=kk),
        grid=(b,),
        in_specs=[
            pl.BlockSpec((1, kk, c), lambda bb: (bb, 0, 0)),
            pl.BlockSpec((1, kk, 8), lambda bb: (bb, 0, 0)),
            pl.BlockSpec((kk, c), lambda bb: (0, 0)),
            pl.BlockSpec((kk, c, c), lambda bb: (0, 0, 0)),
            pl.BlockSpec((1, c), lambda bb: (0, 0)),
        ],
        out_specs=pl.BlockSpec((1, 1, c), lambda bb: (bb, 0, 0)),
        out_shape=jax.ShapeDtypeStruct((b, 1, c), jnp.float32),
        scratch_shapes=[pltpu.VMEM((kk, c), jnp.float32)],
        interpret=_INTERP,
    )(e_acc, sa_acc, cw, w3, lb_row)


def _fin_body(y_ref, s_ref, q_ref, g_ref, b_ref, c_ref, o_ref, *, cnt):
    mean = s_ref[...] / cnt
    var = q_ref[...] / cnt - mean * mean
    inv = lax.rsqrt(var + 1e-5)
    x = jnp.maximum((y_ref[...] - mean) * (inv * g_ref[...]) + b_ref[...], 0.0)
    scaled = x * c_ref[0]
    o_ref[0] = scaled.T


def _finalize(y2, s2, q2, g2, b2, ctx, b, n):
    c = y2.shape[1]
    return pl.pallas_call(
        functools.partial(_fin_body, cnt=float(y2.shape[0])),
        grid=(b, n // _NBO),
        in_specs=[
            pl.BlockSpec((_NBO, c), lambda bb, i: (bb * (n // _NBO) + i, 0)),
            pl.BlockSpec((1, c), lambda bb, i: (0, 0)),
            pl.BlockSpec((1, c), lambda bb, i: (0, 0)),
            pl.BlockSpec((1, c), lambda bb, i: (0, 0)),
            pl.BlockSpec((1, c), lambda bb, i: (0, 0)),
            pl.BlockSpec((1, 1, c), lambda bb, i: (bb, 0, 0)),
        ],
        out_specs=pl.BlockSpec((1, c, _NBO), lambda bb, i: (bb, 0, i)),
        out_shape=jax.ShapeDtypeStruct((b, c, n), jnp.float32),
        interpret=_INTERP,
    )(y2, s2, q2, g2, b2, ctx)


def kernel(unknown, known, unknow_feats, known_feats, conv1_w, bn1_g, bn1_b,
           conv2_w, bn2_g, bn2_b, codewords, scale, lin_w, lin_b):
    b, n, _ = unknown.shape
    m = known.shape[1]
    c1 = unknow_feats.shape[1]
    c2 = known_feats.shape[1]
    dd = conv2_w.shape[0]
    kk = codewords.shape[0]

    known_t = jnp.transpose(known, (0, 2, 1))
    kf_rows = jnp.transpose(known_feats, (0, 2, 1)).reshape(b * m, c2)

    gi, w3 = _three_nn(unknown, known_t)
    interp = _sc_interpolate(kf_rows, gi, w3, b * n, c2)
    p1 = _pre_conv1(unknow_feats, conv1_w[:, c2:].T, b, n)

    y1, s1, q1 = _conv1(interp, p1, conv1_w[:, :c2].T, b, n)
    y2, s2, q2 = _conv2(y1, s1, q1, bn1_g.reshape(1, -1), bn1_b.reshape(1, -1),
                        conv2_w.T)
    cc_row = jnp.sum(codewords * codewords, axis=1).reshape(1, -1)
    g2r = bn2_g.reshape(1, -1)
    b2r = bn2_b.reshape(1, -1)
    e_acc, sa_acc = _enc(y2, s2, q2, g2r, b2r, codewords.T, cc_row,
                         scale.reshape(1, -1), b, n)
    w3lin = lin_w.reshape(dd, kk, dd).transpose(1, 2, 0)
    ctx = _context(e_acc, sa_acc, codewords, w3lin, lin_b.reshape(1, -1))
    return _finalize(y2, s2, q2, g2r, b2r, ctx, b, n)


# speedup vs baseline: 11.9950x; 1.0260x over previous
import functools

import jax
import jax.numpy as jnp
from jax import lax
from jax.experimental import pallas as pl
from jax.experimental.pallas import tpu as pltpu
from jax.experimental.pallas import tpu_sc as plsc

_INTERP = False

_NB1 = 256
_NB = 512
_NBO = 256


def _three_nn_body(u_ref, kt_ref, gi_ref, w_ref, *, m):
    b = pl.program_id(0)
    u = u_ref[0]
    kt = kt_ref[0]
    dx = u[:, 0:1] - kt[0:1, :]
    dy = u[:, 1:2] - kt[1:2, :]
    dz = u[:, 2:3] - kt[2:3, :]
    d = dx * dx + dy * dy + dz * dz
    iota_f = lax.broadcasted_iota(jnp.int32, d.shape, 1).astype(jnp.float32)
    idxs, dists = [], []
    for _ in range(3):
        mj = jnp.min(d, axis=1, keepdims=True)
        eq = d == mj
        ij = jnp.min(jnp.where(eq, iota_f, jnp.float32(m)),
                     axis=1, keepdims=True)
        d = jnp.where(eq, jnp.float32(3.4e38), d)
        idxs.append(ij)
        dists.append(mj)
    dr = [1.0 / (dj + 1e-8) for dj in dists]
    norm = dr[0] + dr[1] + dr[2]
    idx_t = jnp.transpose(jnp.concatenate(idxs, axis=1))
    w_t = jnp.transpose(jnp.concatenate([r / norm for r in dr], axis=1))
    gi_ref[...] = idx_t.astype(jnp.int32) + b * m
    w_ref[...] = w_t


def _three_nn(unknown, known_t):
    b, n, _ = unknown.shape
    m = known_t.shape[2]
    nblk = n // _NB1
    return pl.pallas_call(
        functools.partial(_three_nn_body, m=m),
        grid=(b, nblk),
        in_specs=[
            pl.BlockSpec((1, _NB1, 3), lambda bb, i: (bb, i, 0)),
            pl.BlockSpec((1, 3, m), lambda bb, i: (bb, 0, 0)),
        ],
        out_specs=[
            pl.BlockSpec((3, _NB1), lambda bb, i: (0, bb * nblk + i)),
            pl.BlockSpec((3, _NB1), lambda bb, i: (0, bb * nblk + i)),
        ],
        out_shape=[
            jax.ShapeDtypeStruct((3, b * n), jnp.int32),
            jax.ShapeDtypeStruct((3, b * n), jnp.float32),
        ],
        interpret=_INTERP,
    )(unknown, known_t)


def _sc_interpolate(kf_rows, idx_flat, w_flat, num_pts, c2):
    nw = 32
    ppw = num_pts // nw
    cp = 32
    nchunks = ppw // cp
    mesh = plsc.VectorSubcoreMesh(core_axis_name="c", subcore_axis_name="s")

    @functools.partial(
        pl.kernel,
        mesh=mesh,
        out_type=jax.ShapeDtypeStruct((num_pts, c2), jnp.float32),
        scratch_types=[
            pltpu.VMEM((3, ppw), jnp.int32),
            pltpu.VMEM((3, ppw + 16), jnp.float32),
            pltpu.VMEM((2, 3, cp, c2), jnp.float32),
            pltpu.VMEM((2, cp, c2), jnp.float32),
            pltpu.SemaphoreType.DMA,
            pltpu.SemaphoreType.DMA,
        ],
    )
    def k(kf_hbm, idx_hbm, w_hbm, out_hbm, idx_v, w_v, rows_v, out_v,
          sem_g, sem_o):
        wid = lax.axis_index("s") * 2 + lax.axis_index("c")
        pt_base = wid * ppw
        pltpu.sync_copy(idx_hbm.at[:, pl.ds(pt_base, ppw)], idx_v)
        pltpu.sync_copy(w_hbm.at[:, pl.ds(pt_base, ppw)],
                        w_v.at[:, pl.ds(0, ppw)])

        def start_gather(ci, slot):
            for j in range(3):
                pltpu.async_copy(
                    kf_hbm.at[idx_v.at[j, pl.ds(ci * cp, cp)]],
                    rows_v.at[slot, j], sem_g)

        def wait_gather(ci, slot):
            for j in range(3):
                pltpu.make_async_copy(
                    kf_hbm.at[idx_v.at[j, pl.ds(ci * cp, cp)]],
                    rows_v.at[slot, j], sem_g).wait()

        start_gather(0, 0)

        def pair(cb, carry):
            for b in range(2):
                ci = 2 * cb + b
                wait_gather(ci, b)

                @pl.when(ci + 1 < nchunks)
                def _():
                    start_gather(ci + 1, 1 - b)

                @pl.when(ci >= 2)
                def _():
                    pltpu.make_async_copy(
                        out_v.at[b], out_hbm.at[pl.ds(0, cp)], sem_o).wait()

                for g in range(0, cp, 16):
                    wv0 = w_v[0, pl.ds(ci * cp + g, 16)]
                    wv1 = w_v[1, pl.ds(ci * cp + g, 16)]
                    wv2 = w_v[2, pl.ds(ci * cp + g, 16)]
                    for e in range(16):
                        p = g + e
                        w0 = wv0[e]
                        w1 = wv1[e]
                        w2 = wv2[e]
                        for c in range(c2 // 16):
                            sl = pl.ds(c * 16, 16)
                            out_v[b, p, sl] = (w0 * rows_v[b, 0, p, sl]
                                               + w1 * rows_v[b, 1, p, sl]
                                               + w2 * rows_v[b, 2, p, sl])
                pltpu.async_copy(out_v.at[b],
                                 out_hbm.at[pl.ds(pt_base + ci * cp, cp)],
                                 sem_o)
            return carry

        lax.fori_loop(0, nchunks // 2, pair, 0)
        for b in range(2):
            pltpu.make_async_copy(out_v.at[b], out_hbm.at[pl.ds(0, cp)],
                                  sem_o).wait()

    return k(kf_rows, idx_flat, w_flat)


def _pre_body(uf_ref, wb_ref, p_ref):
    p_ref[...] = lax.dot_general(uf_ref[0], wb_ref[...],
                                 (((0,), (0,)), ((), ())),
                                 preferred_element_type=jnp.float32)


def _pre_conv1(uf, wb_t, b, n):
    c1 = uf.shape[1]
    cout = wb_t.shape[1]
    return pl.pallas_call(
        _pre_body,
        grid=(b, n // _NB),
        in_specs=[
            pl.BlockSpec((1, c1, _NB), lambda bb, i: (bb, 0, i)),
            pl.BlockSpec((c1, cout), lambda bb, i: (0, 0)),
        ],
        out_specs=pl.BlockSpec((_NB, cout),
                               lambda bb, i: (bb * (n // _NB) + i, 0)),
        out_shape=jax.ShapeDtypeStruct((b * n, cout), jnp.float32),
        interpret=_INTERP,
    )(uf, wb_t)


def _conv1_body(x1_ref, p_ref, wa_ref, y_ref, s_ref, q_ref):
    first = jnp.logical_and(pl.program_id(0) == 0, pl.program_id(1) == 0)
    y = (jnp.dot(x1_ref[...], wa_ref[...], preferred_element_type=jnp.float32)
         + p_ref[...])
    y_ref[...] = y.astype(jnp.bfloat16)

    @pl.when(first)
    def _():
        s_ref[...] = jnp.zeros_like(s_ref)
        q_ref[...] = jnp.zeros_like(q_ref)

    s_ref[...] += jnp.sum(y, axis=0, keepdims=True)
    q_ref[...] += jnp.sum(y * y, axis=0, keepdims=True)


def _conv1(interp, p1, wa_t, b, n):
    c2 = interp.shape[1]
    cout = wa_t.shape[1]
    return pl.pallas_call(
        _conv1_body,
        grid=(b, n // _NB),
        in_specs=[
            pl.BlockSpec((_NB, c2), lambda bb, i: (bb * (n // _NB) + i, 0)),
            pl.BlockSpec((_NB, cout), lambda bb, i: (bb * (n // _NB) + i, 0)),
            pl.BlockSpec((c2, cout), lambda bb, i: (0, 0)),
        ],
        out_specs=[
            pl.BlockSpec((_NB, cout), lambda bb, i: (bb * (n // _NB) + i, 0)),
            pl.BlockSpec((1, cout), lambda bb, i: (0, 0)),
            pl.BlockSpec((1, cout), lambda bb, i: (0, 0)),
        ],
        out_shape=[
            jax.ShapeDtypeStruct((b * n, cout), jnp.bfloat16),
            jax.ShapeDtypeStruct((1, cout), jnp.float32),
            jax.ShapeDtypeStruct((1, cout), jnp.float32),
        ],
        interpret=_INTERP,
    )(interp, p1, wa_t)


def _conv2_body(y_ref, s_ref, q_ref, g_ref, b_ref, wc_ref, y2_ref, s2_ref,
                q2_ref, *, cnt):
    i = pl.program_id(0)
    mean = s_ref[...] / cnt
    var = q_ref[...] / cnt - mean * mean
    inv = lax.rsqrt(var + 1e-5)
    y1 = y_ref[...].astype(jnp.float32)
    z = jnp.maximum((y1 - mean) * (inv * g_ref[...]) + b_ref[...], 0.0)
    y2 = jnp.dot(z, wc_ref[...], preferred_element_type=jnp.float32)
    y2_ref[...] = y2.astype(jnp.bfloat16)

    @pl.when(i == 0)
    def _():
        s2_ref[...] = jnp.zeros_like(s2_ref)
        q2_ref[...] = jnp.zeros_like(q2_ref)

    s2_ref[...] += jnp.sum(y2, axis=0, keepdims=True)
    q2_ref[...] += jnp.sum(y2 * y2, axis=0, keepdims=True)


def _conv2(y1, s1, q1, g1, b1, wc_t):
    bn_, c = y1.shape
    cout = wc_t.shape[1]
    return pl.pallas_call(
        functools.partial(_conv2_body, cnt=float(bn_)),
        grid=(bn_ // _NB,),
        in_specs=[
            pl.BlockSpec((_NB, c), lambda i: (i, 0)),
            pl.BlockSpec((1, c), lambda i: (0, 0)),
            pl.BlockSpec((1, c), lambda i: (0, 0)),
            pl.BlockSpec((1, c), lambda i: (0, 0)),
            pl.BlockSpec((1, c), lambda i: (0, 0)),
            pl.BlockSpec((c, cout), lambda i: (0, 0)),
        ],
        out_specs=[
            pl.BlockSpec((_NB, cout), lambda i: (i, 0)),
            pl.BlockSpec((1, cout), lambda i: (0, 0)),
            pl.BlockSpec((1, cout), lambda i: (0, 0)),
        ],
        out_shape=[
            jax.ShapeDtypeStruct((bn_, cout), jnp.bfloat16),
            jax.ShapeDtypeStruct((1, cout), jnp.float32),
            jax.ShapeDtypeStruct((1, cout), jnp.float32),
        ],
        interpret=_INTERP,
    )(y1, s1, q1, g1, b1, wc_t)


def _enc_body(y_ref, s_ref, q_ref, g_ref, b_ref, cwt_ref, cc_ref, sc_ref,
              e_ref, sa_ref, *, cnt):
    i = pl.program_id(1)
    mean = s_ref[...] / cnt
    var = q_ref[...] / cnt - mean * mean
    inv = lax.rsqrt(var + 1e-5)
    y2 = y_ref[...].astype(jnp.float32)
    x = jnp.maximum((y2 - mean) * (inv * g_ref[...]) + b_ref[...], 0.0)
    xx = jnp.sum(x * x, axis=1, keepdims=True)
    xc = jnp.dot(x, cwt_ref[...], preferred_element_type=jnp.float32)
    sl = sc_ref[...] * (xx + cc_ref[...] - 2.0 * xc)
    mx = jnp.max(sl, axis=1, keepdims=True)
    ea = jnp.exp(sl - mx)
    a = ea / jnp.sum(ea, axis=1, keepdims=True)
    ep = lax.dot_general(a, x, (((0,), (0,)), ((), ())),
                         preferred_element_type=jnp.float32)
    ones = jnp.ones((a.shape[0], 8), jnp.float32)
    sap = lax.dot_general(a, ones, (((0,), (0,)), ((), ())),
                          preferred_element_type=jnp.float32)

    @pl.when(i == 0)
    def _():
        e_ref[...] = jnp.zeros_like(e_ref)
        sa_ref[...] = jnp.zeros_like(sa_ref)

    e_ref[...] += ep[None]
    sa_ref[...] += sap[None]


def _enc(y2, s2, q2, g2, b2, cw_t, cc_row, sc_row, b, n):
    c = y2.shape[1]
    kk = cw_t.shape[1]
    return pl.pallas_call(
        functools.partial(_enc_body, cnt=float(y2.shape[0])),
        grid=(b, n // _NB),
        in_specs=[
            pl.BlockSpec((_NB, c), lambda bb, i: (bb * (n // _NB) + i, 0)),
            pl.BlockSpec((1, c), lambda bb, i: (0, 0)),
            pl.BlockSpec((1, c), lambda bb, i: (0, 0)),
            pl.BlockSpec((1, c), lambda bb, i: (0, 0)),
            pl.BlockSpec((1, c), lambda bb, i: (0, 0)),
            pl.BlockSpec((c, kk), lambda bb, i: (0, 0)),
            pl.BlockSpec((1, kk), lambda bb, i: (0, 0)),
            pl.BlockSpec((1, kk), lambda bb, i: (0, 0)),
        ],
        out_specs=[
            pl.BlockSpec((1, kk, c), lambda bb, i: (bb, 0, 0)),
            pl.BlockSpec((1, kk, 8), lambda bb, i: (bb, 0, 0)),
        ],
        out_shape=[
            jax.ShapeDtypeStruct((b, kk, c), jnp.float32),
            jax.ShapeDtypeStruct((b, kk, 8), jnp.float32),
        ],
        interpret=_INTERP,
    )(y2, s2, q2, g2, b2, cw_t, cc_row, sc_row)


def _ctx_body(e_ref, sa_ref, cw_ref, w3_ref, lb_ref, o_ref, es_ref, *, kk):
    e = jnp.maximum(e_ref[0] - sa_ref[0][:, 0:1] * cw_ref[...], 0.0)
    es_ref[...] = e
    nrm = jnp.sqrt(jnp.sum(e * e))
    sc = 1.0 / jnp.maximum(nrm, 1e-12)

    def kb(k, acc):
        ek = es_ref[pl.ds(k, 1), :]
        wk = w3_ref[k]
        return acc + jnp.dot(ek, wk, preferred_element_type=jnp.float32)

    acc = lax.fori_loop(0, kk, kb,
                        jnp.zeros((1, e.shape[1]), jnp.float32))
    z = acc * sc + lb_ref[...]
    o_ref[0] = 1.0 / (1.0 + jnp.exp(-z))


def _context(e_acc, sa_acc, cw, w3, lb_row):
    b, kk, c = e_acc.shape
    return pl.pallas_call(
        functools.partial(_ctx_body, kk=kk),
        grid=(b,),
        in_specs=[
            pl.BlockSpec((1, kk, c), lambda bb: (bb, 0, 0)),
            pl.BlockSpec((1, kk, 8), lambda bb: (bb, 0, 0)),
            pl.BlockSpec((kk, c), lambda bb: (0, 0)),
            pl.BlockSpec((kk, c, c), lambda bb: (0, 0, 0)),
            pl.BlockSpec((1, c), lambda bb: (0, 0)),
        ],
        out_specs=pl.BlockSpec((1, 1, c), lambda bb: (bb, 0, 0)),
        out_shape=jax.ShapeDtypeStruct((b, 1, c), jnp.float32),
        scratch_shapes=[pltpu.VMEM((kk, c), jnp.float32)],
        interpret=_INTERP,
    )(e_acc, sa_acc, cw, w3, lb_row)


def _fin_body(y_ref, s_ref, q_ref, g_ref, b_ref, c_ref, o_ref, *, cnt):
    mean = s_ref[...] / cnt
    var = q_ref[...] / cnt - mean * mean
    inv = lax.rsqrt(var + 1e-5)
    y2 = y_ref[...].astype(jnp.float32)
    x = jnp.maximum((y2 - mean) * (inv * g_ref[...]) + b_ref[...], 0.0)
    scaled = x * c_ref[0]
    o_ref[0] = scaled.T


def _finalize(y2, s2, q2, g2, b2, ctx, b, n):
    c = y2.shape[1]
    return pl.pallas_call(
        functools.partial(_fin_body, cnt=float(y2.shape[0])),
        grid=(b, n // _NBO),
        in_specs=[
            pl.BlockSpec((_NBO, c), lambda bb, i: (bb * (n // _NBO) + i, 0)),
            pl.BlockSpec((1, c), lambda bb, i: (0, 0)),
            pl.BlockSpec((1, c), lambda bb, i: (0, 0)),
            pl.BlockSpec((1, c), lambda bb, i: (0, 0)),
            pl.BlockSpec((1, c), lambda bb, i: (0, 0)),
            pl.BlockSpec((1, 1, c), lambda bb, i: (bb, 0, 0)),
        ],
        out_specs=pl.BlockSpec((1, c, _NBO), lambda bb, i: (bb, 0, i)),
        out_shape=jax.ShapeDtypeStruct((b, c, n), jnp.float32),
        interpret=_INTERP,
    )(y2, s2, q2, g2, b2, ctx)


def kernel(unknown, known, unknow_feats, known_feats, conv1_w, bn1_g, bn1_b,
           conv2_w, bn2_g, bn2_b, codewords, scale, lin_w, lin_b):
    b, n, _ = unknown.shape
    m = known.shape[1]
    c1 = unknow_feats.shape[1]
    c2 = known_feats.shape[1]
    dd = conv2_w.shape[0]
    kk = codewords.shape[0]

    known_t = jnp.transpose(known, (0, 2, 1))
    kf_rows = jnp.transpose(known_feats, (0, 2, 1)).reshape(b * m, c2)

    gi, w3 = _three_nn(unknown, known_t)
    interp = _sc_interpolate(kf_rows, gi, w3, b * n, c2)
    p1 = _pre_conv1(unknow_feats, conv1_w[:, c2:].T, b, n)

    y1, s1, q1 = _conv1(interp, p1, conv1_w[:, :c2].T, b, n)
    y2, s2, q2 = _conv2(y1, s1, q1, bn1_g.reshape(1, -1), bn1_b.reshape(1, -1),
                        conv2_w.T)
    cc_row = jnp.sum(codewords * codewords, axis=1).reshape(1, -1)
    g2r = bn2_g.reshape(1, -1)
    b2r = bn2_b.reshape(1, -1)
    e_acc, sa_acc = _enc(y2, s2, q2, g2r, b2r, codewords.T, cc_row,
                         scale.reshape(1, -1), b, n)
    w3lin = lin_w.reshape(dd, kk, dd).transpose(1, 2, 0)
    ctx = _context(e_acc, sa_acc, codewords, w3lin, lin_b.reshape(1, -1))
    return _finalize(y2, s2, q2, g2r, b2r, ctx, b, n)


# speedup vs baseline: 12.9497x; 1.0796x over previous
import functools

import jax
import jax.numpy as jnp
from jax import lax
from jax.experimental import pallas as pl
from jax.experimental.pallas import tpu as pltpu
from jax.experimental.pallas import tpu_sc as plsc

_INTERP = False

_NB1 = 256
_NB = 512
_NBO = 256


def _three_nn_body(u_ref, kt_ref, gi_ref, w_ref, *, m, boff):
    b = pl.program_id(0) + boff
    u = u_ref[0]
    kt = kt_ref[0]
    dx = u[:, 0:1] - kt[0:1, :]
    dy = u[:, 1:2] - kt[1:2, :]
    dz = u[:, 2:3] - kt[2:3, :]
    d = dx * dx + dy * dy + dz * dz
    iota_f = lax.broadcasted_iota(jnp.int32, d.shape, 1).astype(jnp.float32)
    idxs, dists = [], []
    for _ in range(3):
        mj = jnp.min(d, axis=1, keepdims=True)
        eq = d == mj
        ij = jnp.min(jnp.where(eq, iota_f, jnp.float32(m)),
                     axis=1, keepdims=True)
        d = jnp.where(eq, jnp.float32(3.4e38), d)
        idxs.append(ij)
        dists.append(mj)
    dr = [1.0 / (dj + 1e-8) for dj in dists]
    norm = dr[0] + dr[1] + dr[2]
    idx_t = jnp.transpose(jnp.concatenate(idxs, axis=1))
    w_t = jnp.transpose(jnp.concatenate([r / norm for r in dr], axis=1))
    gi_ref[...] = idx_t.astype(jnp.int32) + b * m
    w_ref[...] = w_t


def _three_nn(unknown, known_t, boff=0):
    b, n, _ = unknown.shape
    m = known_t.shape[2]
    nblk = n // _NB1
    return pl.pallas_call(
        functools.partial(_three_nn_body, m=m, boff=boff),
        grid=(b, nblk),
        in_specs=[
            pl.BlockSpec((1, _NB1, 3), lambda bb, i: (bb, i, 0)),
            pl.BlockSpec((1, 3, m), lambda bb, i: (bb, 0, 0)),
        ],
        out_specs=[
            pl.BlockSpec((3, _NB1), lambda bb, i: (0, bb * nblk + i)),
            pl.BlockSpec((3, _NB1), lambda bb, i: (0, bb * nblk + i)),
        ],
        out_shape=[
            jax.ShapeDtypeStruct((3, b * n), jnp.int32),
            jax.ShapeDtypeStruct((3, b * n), jnp.float32),
        ],
        interpret=_INTERP,
    )(unknown, known_t)


def _sc_interpolate(kf_rows, idx_flat, w_flat, num_pts, c2):
    nw = 32
    ppw = num_pts // nw
    cp = 32
    nchunks = ppw // cp
    mesh = plsc.VectorSubcoreMesh(core_axis_name="c", subcore_axis_name="s")

    @functools.partial(
        pl.kernel,
        mesh=mesh,
        out_type=jax.ShapeDtypeStruct((num_pts, c2), jnp.float32),
        scratch_types=[
            pltpu.VMEM((3, ppw), jnp.int32),
            pltpu.VMEM((3, ppw + 16), jnp.float32),
            pltpu.VMEM((2, 3, cp, c2), jnp.float32),
            pltpu.VMEM((2, cp, c2), jnp.float32),
            pltpu.SemaphoreType.DMA,
            pltpu.SemaphoreType.DMA,
        ],
    )
    def k(kf_hbm, idx_hbm, w_hbm, out_hbm, idx_v, w_v, rows_v, out_v,
          sem_g, sem_o):
        wid = lax.axis_index("s") * 2 + lax.axis_index("c")
        pt_base = wid * ppw
        pltpu.sync_copy(idx_hbm.at[:, pl.ds(pt_base, ppw)], idx_v)
        pltpu.sync_copy(w_hbm.at[:, pl.ds(pt_base, ppw)],
                        w_v.at[:, pl.ds(0, ppw)])

        def start_gather(ci, slot):
            for j in range(3):
                pltpu.async_copy(
                    kf_hbm.at[idx_v.at[j, pl.ds(ci * cp, cp)]],
                    rows_v.at[slot, j], sem_g)

        def wait_gather(ci, slot):
            for j in range(3):
                pltpu.make_async_copy(
                    kf_hbm.at[idx_v.at[j, pl.ds(ci * cp, cp)]],
                    rows_v.at[slot, j], sem_g).wait()

        start_gather(0, 0)

        def pair(cb, carry):
            for b in range(2):
                ci = 2 * cb + b
                wait_gather(ci, b)

                @pl.when(ci + 1 < nchunks)
                def _():
                    start_gather(ci + 1, 1 - b)

                @pl.when(ci >= 2)
                def _():
                    pltpu.make_async_copy(
                        out_v.at[b], out_hbm.at[pl.ds(0, cp)], sem_o).wait()

                for g in range(0, cp, 16):
                    wv0 = w_v[0, pl.ds(ci * cp + g, 16)]
                    wv1 = w_v[1, pl.ds(ci * cp + g, 16)]
                    wv2 = w_v[2, pl.ds(ci * cp + g, 16)]
                    for e in range(16):
                        p = g + e
                        w0 = wv0[e]
                        w1 = wv1[e]
                        w2 = wv2[e]
                        for c in range(c2 // 16):
                            sl = pl.ds(c * 16, 16)
                            out_v[b, p, sl] = (w0 * rows_v[b, 0, p, sl]
                                               + w1 * rows_v[b, 1, p, sl]
                                               + w2 * rows_v[b, 2, p, sl])
                pltpu.async_copy(out_v.at[b],
                                 out_hbm.at[pl.ds(pt_base + ci * cp, cp)],
                                 sem_o)
            return carry

        lax.fori_loop(0, nchunks // 2, pair, 0)
        for b in range(2):
            pltpu.make_async_copy(out_v.at[b], out_hbm.at[pl.ds(0, cp)],
                                  sem_o).wait()

    return k(kf_rows, idx_flat, w_flat)


def _pre_body(uf_ref, wb_ref, p_ref):
    p_ref[...] = lax.dot_general(uf_ref[0], wb_ref[...],
                                 (((0,), (0,)), ((), ())),
                                 preferred_element_type=jnp.float32)


def _pre_conv1(uf, wb_t, b, n):
    c1 = uf.shape[1]
    cout = wb_t.shape[1]
    return pl.pallas_call(
        _pre_body,
        grid=(b, n // _NB),
        in_specs=[
            pl.BlockSpec((1, c1, _NB), lambda bb, i: (bb, 0, i)),
            pl.BlockSpec((c1, cout), lambda bb, i: (0, 0)),
        ],
        out_specs=pl.BlockSpec((_NB, cout),
                               lambda bb, i: (bb * (n // _NB) + i, 0)),
        out_shape=jax.ShapeDtypeStruct((b * n, cout), jnp.float32),
        interpret=_INTERP,
    )(uf, wb_t)


def _conv1_body(xa_ref, xb_ref, p_ref, wa_ref, y_ref, s_ref, q_ref):
    h = pl.program_id(0)
    first = jnp.logical_and(h == 0, pl.program_id(1) == 0)
    x1 = jnp.where(h == 0, xa_ref[...], xb_ref[...])
    y = (jnp.dot(x1, wa_ref[...], preferred_element_type=jnp.float32)
         + p_ref[...])
    y_ref[...] = y.astype(jnp.bfloat16)

    @pl.when(first)
    def _():
        s_ref[...] = jnp.zeros_like(s_ref)
        q_ref[...] = jnp.zeros_like(q_ref)

    s_ref[...] += jnp.sum(y, axis=0, keepdims=True)
    q_ref[...] += jnp.sum(y * y, axis=0, keepdims=True)


def _conv1(interp_a, interp_b, p1, wa_t, b, n):
    c2 = interp_a.shape[1]
    cout = wa_t.shape[1]
    nhb = (b * n) // (2 * _NB)
    return pl.pallas_call(
        _conv1_body,
        grid=(2, nhb),
        in_specs=[
            pl.BlockSpec((_NB, c2), lambda h, i: ((1 - h) * i, 0)),
            pl.BlockSpec((_NB, c2), lambda h, i: (h * i, 0)),
            pl.BlockSpec((_NB, cout), lambda h, i: (h * nhb + i, 0)),
            pl.BlockSpec((c2, cout), lambda h, i: (0, 0)),
        ],
        out_specs=[
            pl.BlockSpec((_NB, cout), lambda h, i: (h * nhb + i, 0)),
            pl.BlockSpec((1, cout), lambda h, i: (0, 0)),
            pl.BlockSpec((1, cout), lambda h, i: (0, 0)),
        ],
        out_shape=[
            jax.ShapeDtypeStruct((b * n, cout), jnp.bfloat16),
            jax.ShapeDtypeStruct((1, cout), jnp.float32),
            jax.ShapeDtypeStruct((1, cout), jnp.float32),
        ],
        interpret=_INTERP,
    )(interp_a, interp_b, p1, wa_t)


def _conv2_body(y_ref, s_ref, q_ref, g_ref, b_ref, wc_ref, y2_ref, s2_ref,
                q2_ref, *, cnt):
    i = pl.program_id(0)
    mean = s_ref[...] / cnt
    var = q_ref[...] / cnt - mean * mean
    inv = lax.rsqrt(var + 1e-5)
    y1 = y_ref[...].astype(jnp.float32)
    z = jnp.maximum((y1 - mean) * (inv * g_ref[...]) + b_ref[...], 0.0)
    y2 = jnp.dot(z, wc_ref[...], preferred_element_type=jnp.float32)
    y2_ref[...] = y2.astype(jnp.bfloat16)

    @pl.when(i == 0)
    def _():
        s2_ref[...] = jnp.zeros_like(s2_ref)
        q2_ref[...] = jnp.zeros_like(q2_ref)

    s2_ref[...] += jnp.sum(y2, axis=0, keepdims=True)
    q2_ref[...] += jnp.sum(y2 * y2, axis=0, keepdims=True)


def _conv2(y1, s1, q1, g1, b1, wc_t):
    bn_, c = y1.shape
    cout = wc_t.shape[1]
    return pl.pallas_call(
        functools.partial(_conv2_body, cnt=float(bn_)),
        grid=(bn_ // _NB,),
        in_specs=[
            pl.BlockSpec((_NB, c), lambda i: (i, 0)),
            pl.BlockSpec((1, c), lambda i: (0, 0)),
            pl.BlockSpec((1, c), lambda i: (0, 0)),
            pl.BlockSpec((1, c), lambda i: (0, 0)),
            pl.BlockSpec((1, c), lambda i: (0, 0)),
            pl.BlockSpec((c, cout), lambda i: (0, 0)),
        ],
        out_specs=[
            pl.BlockSpec((_NB, cout), lambda i: (i, 0)),
            pl.BlockSpec((1, cout), lambda i: (0, 0)),
            pl.BlockSpec((1, cout), lambda i: (0, 0)),
        ],
        out_shape=[
            jax.ShapeDtypeStruct((bn_, cout), jnp.bfloat16),
            jax.ShapeDtypeStruct((1, cout), jnp.float32),
            jax.ShapeDtypeStruct((1, cout), jnp.float32),
        ],
        interpret=_INTERP,
    )(y1, s1, q1, g1, b1, wc_t)


def _enc_body(y_ref, s_ref, q_ref, g_ref, b_ref, cwt_ref, cc_ref, sc_ref,
              e_ref, sa_ref, *, cnt):
    i = pl.program_id(1)
    mean = s_ref[...] / cnt
    var = q_ref[...] / cnt - mean * mean
    inv = lax.rsqrt(var + 1e-5)
    y2 = y_ref[...].astype(jnp.float32)
    x = jnp.maximum((y2 - mean) * (inv * g_ref[...]) + b_ref[...], 0.0)
    xx = jnp.sum(x * x, axis=1, keepdims=True)
    xc = jnp.dot(x, cwt_ref[...], preferred_element_type=jnp.float32)
    sl = sc_ref[...] * (xx + cc_ref[...] - 2.0 * xc)
    mx = jnp.max(sl, axis=1, keepdims=True)
    ea = jnp.exp(sl - mx)
    a = ea / jnp.sum(ea, axis=1, keepdims=True)
    ep = lax.dot_general(a, x, (((0,), (0,)), ((), ())),
                         preferred_element_type=jnp.float32)
    ones = jnp.ones((a.shape[0], 8), jnp.float32)
    sap = lax.dot_general(a, ones, (((0,), (0,)), ((), ())),
                          preferred_element_type=jnp.float32)

    @pl.when(i == 0)
    def _():
        e_ref[...] = jnp.zeros_like(e_ref)
        sa_ref[...] = jnp.zeros_like(sa_ref)

    e_ref[...] += ep[None]
    sa_ref[...] += sap[None]


def _enc(y2, s2, q2, g2, b2, cw_t, cc_row, sc_row, b, n):
    c = y2.shape[1]
    kk = cw_t.shape[1]
    return pl.pallas_call(
        functools.partial(_enc_body, cnt=float(y2.shape[0])),
        grid=(b, n // _NB),
        in_specs=[
            pl.BlockSpec((_NB, c), lambda bb, i: (bb * (n // _NB) + i, 0)),
            pl.BlockSpec((1, c), lambda bb, i: (0, 0)),
            pl.BlockSpec((1, c), lambda bb, i: (0, 0)),
            pl.BlockSpec((1, c), lambda bb, i: (0, 0)),
            pl.BlockSpec((1, c), lambda bb, i: (0, 0)),
            pl.BlockSpec((c, kk), lambda bb, i: (0, 0)),
            pl.BlockSpec((1, kk), lambda bb, i: (0, 0)),
            pl.BlockSpec((1, kk), lambda bb, i: (0, 0)),
        ],
        out_specs=[
            pl.BlockSpec((1, kk, c), lambda bb, i: (bb, 0, 0)),
            pl.BlockSpec((1, kk, 8), lambda bb, i: (bb, 0, 0)),
        ],
        out_shape=[
            jax.ShapeDtypeStruct((b, kk, c), jnp.float32),
            jax.ShapeDtypeStruct((b, kk, 8), jnp.float32),
        ],
        interpret=_INTERP,
    )(y2, s2, q2, g2, b2, cw_t, cc_row, sc_row)


def _ctx_body(e_ref, sa_ref, cw_ref, w3_ref, lb_ref, o_ref, es_ref, *, kk):
    e = jnp.maximum(e_ref[0] - sa_ref[0][:, 0:1] * cw_ref[...], 0.0)
    es_ref[...] = e
    nrm = jnp.sqrt(jnp.sum(e * e))
    sc = 1.0 / jnp.maximum(nrm, 1e-12)

    def kb(k, acc):
        ek = es_ref[pl.ds(k, 1), :]
        wk = w3_ref[k]
        return acc + jnp.dot(ek, wk, preferred_element_type=jnp.float32)

    acc = lax.fori_loop(0, kk, kb,
                        jnp.zeros((1, e.shape[1]), jnp.float32))
    z = acc * sc + lb_ref[...]
    o_ref[0] = 1.0 / (1.0 + jnp.exp(-z))


def _context(e_acc, sa_acc, cw, w3, lb_row):
    b, kk, c = e_acc.shape
    return pl.pallas_call(
        functools.partial(_ctx_body, kk=kk),
        grid=(b,),
        in_specs=[
            pl.BlockSpec((1, kk, c), lambda bb: (bb, 0, 0)),
            pl.BlockSpec((1, kk, 8), lambda bb: (bb, 0, 0)),
            pl.BlockSpec((kk, c), lambda bb: (0, 0)),
            pl.BlockSpec((kk, c, c), lambda bb: (0, 0, 0)),
            pl.BlockSpec((1, c), lambda bb: (0, 0)),
        ],
        out_specs=pl.BlockSpec((1, 1, c), lambda bb: (bb, 0, 0)),
        out_shape=jax.ShapeDtypeStruct((b, 1, c), jnp.float32),
        scratch_shapes=[pltpu.VMEM((kk, c), jnp.float32)],
        interpret=_INTERP,
    )(e_acc, sa_acc, cw, w3, lb_row)


def _fin_body(y_ref, s_ref, q_ref, g_ref, b_ref, c_ref, o_ref, *, cnt):
    mean = s_ref[...] / cnt
    var = q_ref[...] / cnt - mean * mean
    inv = lax.rsqrt(var + 1e-5)
    y2 = y_ref[...].astype(jnp.float32)
    x = jnp.maximum((y2 - mean) * (inv * g_ref[...]) + b_ref[...], 0.0)
    scaled = x * c_ref[0]
    o_ref[0] = scaled.T


def _finalize(y2, s2, q2, g2, b2, ctx, b, n):
    c = y2.shape[1]
    return pl.pallas_call(
        functools.partial(_fin_body, cnt=float(y2.shape[0])),
        grid=(b, n // _NBO),
        in_specs=[
            pl.BlockSpec((_NBO, c), lambda bb, i: (bb * (n // _NBO) + i, 0)),
            pl.BlockSpec((1, c), lambda bb, i: (0, 0)),
            pl.BlockSpec((1, c), lambda bb, i: (0, 0)),
            pl.BlockSpec((1, c), lambda bb, i: (0, 0)),
            pl.BlockSpec((1, c), lambda bb, i: (0, 0)),
            pl.BlockSpec((1, 1, c), lambda bb, i: (bb, 0, 0)),
        ],
        out_specs=pl.BlockSpec((1, c, _NBO), lambda bb, i: (bb, 0, i)),
        out_shape=jax.ShapeDtypeStruct((b, c, n), jnp.float32),
        interpret=_INTERP,
    )(y2, s2, q2, g2, b2, ctx)


def kernel(unknown, known, unknow_feats, known_feats, conv1_w, bn1_g, bn1_b,
           conv2_w, bn2_g, bn2_b, codewords, scale, lin_w, lin_b):
    b, n, _ = unknown.shape
    m = known.shape[1]
    c1 = unknow_feats.shape[1]
    c2 = known_feats.shape[1]
    dd = conv2_w.shape[0]
    kk = codewords.shape[0]

    known_t = jnp.transpose(known, (0, 2, 1))
    kf_rows = jnp.transpose(known_feats, (0, 2, 1)).reshape(b * m, c2)

    bh = b // 2
    gia, wta = _three_nn(unknown[:bh], known_t[:bh], 0)
    interp_a = _sc_interpolate(kf_rows, gia, wta, bh * n, c2)
    gib, wtb = _three_nn(unknown[bh:], known_t[bh:], bh)
    interp_b = _sc_interpolate(kf_rows, gib, wtb, bh * n, c2)
    p1 = _pre_conv1(unknow_feats, conv1_w[:, c2:].T, b, n)

    y1, s1, q1 = _conv1(interp_a, interp_b, p1, conv1_w[:, :c2].T, b, n)
    y2, s2, q2 = _conv2(y1, s1, q1, bn1_g.reshape(1, -1), bn1_b.reshape(1, -1),
                        conv2_w.T)
    cc_row = jnp.sum(codewords * codewords, axis=1).reshape(1, -1)
    g2r = bn2_g.reshape(1, -1)
    b2r = bn2_b.reshape(1, -1)
    e_acc, sa_acc = _enc(y2, s2, q2, g2r, b2r, codewords.T, cc_row,
                         scale.reshape(1, -1), b, n)
    w3lin = lin_w.reshape(dd, kk, dd).transpose(1, 2, 0)
    ctx = _context(e_acc, sa_acc, codewords, w3lin, lin_b.reshape(1, -1))
    return _finalize(y2, s2, q2, g2r, b2r, ctx, b, n)
